# Initial kernel scaffold; baseline (speedup 1.0000x reference)
import jax
import jax.numpy as jnp
from jax.experimental import pallas as pl


def kernel(x, weight, bias):
    raise NotImplementedError("write your pallas kernel here")


# speedup vs baseline: 1.9939x; 1.9939x over previous
import functools

import jax
import jax.numpy as jnp
from jax.experimental import pallas as pl

_N_GROUPS = 8
_TOPK_GROUPS = 4
_TOPK = 8
_ROUTE_SCALE = 2.5


def _router_body(x_ref, w_ref, b_ref, wout_ref, iout_ref, *, tt, e):
    gs_sz = e // _N_GROUPS
    xb = x_ref[...]
    w = w_ref[...]
    logits = jax.lax.dot_general(
        xb, w, dimension_numbers=(((1,), (1,)), ((), ())),
        preferred_element_type=jnp.float32)
    orig = jax.nn.sigmoid(logits)
    s = orig + b_ref[...]

    gs_cols = []
    for g in range(_N_GROUPS):
        sg = s[:, g * gs_sz:(g + 1) * gs_sz]
        m1 = jnp.max(sg, axis=1, keepdims=True)
        eq = sg == m1
        cnt = jnp.sum(eq.astype(jnp.float32), axis=1, keepdims=True)
        m2p = jnp.max(jnp.where(eq, -jnp.inf, sg), axis=1, keepdims=True)
        m2 = jnp.where(cnt >= 2.0, m1, m2p)
        gs_cols.append(m1 + m2)

    neg = jnp.float32(-jnp.inf)
    sm_cols = []
    for g in range(_N_GROUPS):
        r = jnp.zeros((tt, 1), jnp.float32)
        for h in range(_N_GROUPS):
            if h == g:
                continue
            c = (gs_cols[h] >= gs_cols[g]) if h < g else (gs_cols[h] > gs_cols[g])
            r = r + c.astype(jnp.float32)
        sel = r < float(_TOPK_GROUPS)
        sm_cols.append(jnp.where(sel, s[:, g * gs_sz:(g + 1) * gs_sz], neg))
    sm = jnp.concatenate(sm_cols, axis=1)
    iota = jax.lax.broadcasted_iota(jnp.int32, (tt, e), 1)

    idx_cols, w_cols = [], []
    for r in range(_TOPK):
        m = jnp.max(sm, axis=1, keepdims=True)
        hit = sm == m
        idx = jnp.min(jnp.where(hit, iota, e), axis=1, keepdims=True)
        sel1 = iota == idx
        w_cols.append(jnp.sum(jnp.where(sel1, orig, 0.0), axis=1, keepdims=True))
        idx_cols.append(idx)
        if r < _TOPK - 1:
            sm = jnp.where(sel1, neg, sm)

    wt = jnp.concatenate(w_cols, axis=1)
    wsum = jnp.sum(wt, axis=1, keepdims=True)
    wt = (wt / wsum) * _ROUTE_SCALE
    wout_ref[...] = wt
    iout_ref[...] = jnp.concatenate(idx_cols, axis=1)


@jax.jit
def kernel(x, weight, bias):
    t, dim = x.shape
    e = weight.shape[0]
    w_pad = jnp.pad(weight, ((0, 0), (1, 0)))
    bias2 = bias.reshape(1, e).astype(jnp.float32)
    tt = 512
    grid = (t // tt,)
    wt, idx = pl.pallas_call(
        functools.partial(_router_body, tt=tt, e=e),
        grid=grid,
        in_specs=[
            pl.BlockSpec((tt, dim), lambda i: (i, 0)),
            pl.BlockSpec((e, dim), lambda i: (0, 0)),
            pl.BlockSpec((1, e), lambda i: (0, 0)),
        ],
        out_specs=[
            pl.BlockSpec((tt, _TOPK), lambda i: (i, 0)),
            pl.BlockSpec((tt, _TOPK), lambda i: (i, 0)),
        ],
        out_shape=[
            jax.ShapeDtypeStruct((t, _TOPK), jnp.float32),
            jax.ShapeDtypeStruct((t, _TOPK), jnp.int32),
        ],
    )(x, w_pad, bias2)
    return wt, idx


# speedup vs baseline: 5.3599x; 2.6881x over previous
import functools

import jax
import jax.numpy as jnp
from jax import lax
from jax.experimental import pallas as pl
from jax.experimental.pallas import tpu as pltpu
from jax.experimental.pallas import tpu_sc as plsc

_E = 64
_N_GROUPS = 8
_GS = _E // _N_GROUPS
_TOPK_GROUPS = 4
_TOPK = 8
_ROUTE_SCALE = 2.5

_NC = 2
_NS = 16
_NW = _NC * _NS
_L = 16


def _scores_body(x_ref, w_ref, b_ref, out_ref):
    logits = jax.lax.dot_general(
        w_ref[...], x_ref[...], dimension_numbers=(((1,), (1,)), ((), ())),
        preferred_element_type=jnp.float32)
    out_ref[...] = jax.nn.sigmoid(logits) + b_ref[...]


def _scores_t(x, w_pad, bias_col):
    t, dim = x.shape
    tt = 1024
    return pl.pallas_call(
        _scores_body,
        grid=(t // tt,),
        in_specs=[
            pl.BlockSpec((tt, dim), lambda i: (i, 0)),
            pl.BlockSpec((_E, dim), lambda i: (0, 0)),
            pl.BlockSpec((_E, 1), lambda i: (0, 0)),
        ],
        out_specs=pl.BlockSpec((_E, tt), lambda i: (0, i)),
        out_shape=jax.ShapeDtypeStruct((_E, t), jnp.float32),
    )(x, w_pad, bias_col)


def _vmax(a, b):
    return jnp.maximum(a, b)


def _tree(op, xs):
    xs = list(xs)
    while len(xs) > 1:
        nxt = [op(xs[i], xs[i + 1]) for i in range(0, len(xs) - 1, 2)]
        if len(xs) % 2:
            nxt.append(xs[-1])
        xs = nxt
    return xs[0]


def _route_chunk(v, bias_ref):
    neg = jnp.full((_L,), -jnp.inf, jnp.float32)
    one = jnp.full((_L,), 1.0, jnp.float32)
    zero = jnp.full((_L,), 0.0, jnp.float32)

    t2 = []
    for g in range(_N_GROUPS):
        vs = v[g * _GS:(g + 1) * _GS]
        m1 = _tree(_vmax, vs)
        eqs = [vs_i == m1 for vs_i in vs]
        cnt = _tree(jnp.add, [jnp.where(e, one, zero) for e in eqs])
        m2p = _tree(_vmax, [jnp.where(e, neg, vs_i)
                            for e, vs_i in zip(eqs, vs)])
        m2 = jnp.where(cnt >= 2.0, m1, m2p)
        t2.append(m1 + m2)

    sel = []
    for g in range(_N_GROUPS):
        terms = []
        for h in range(_N_GROUPS):
            if h == g:
                continue
            c = (t2[h] >= t2[g]) if h < g else (t2[h] > t2[g])
            terms.append(jnp.where(c, one, zero))
        rk = _tree(jnp.add, terms)
        sel.append(rk < float(_TOPK_GROUPS))

    sm = [jnp.where(sel[e // _GS], v[e], neg) for e in range(_E)]
    e_const = [jnp.full((_L,), e, jnp.int32) for e in range(_E)]
    big = jnp.full((_L,), _E, jnp.int32)

    widx, wraw = [], []
    for r in range(_TOPK):
        m = _tree(_vmax, sm)
        cand = [jnp.where(sm[e] == m, e_const[e], big) for e in range(_E)]
        idx = _tree(jnp.minimum, cand)
        bias_at = plsc.load_gather(bias_ref, [idx])
        widx.append(idx)
        wraw.append(m - bias_at)
        if r < _TOPK - 1:
            sm = [jnp.where(idx == e_const[e], neg, sm[e]) for e in range(_E)]

    wsum = _tree(jnp.add, wraw)
    wvals = [(w / wsum) * _ROUTE_SCALE for w in wraw]
    return wvals, widx


def _route_body(sT, bias_hbm, wout, iout,
                span_v, wspan_v, ispan_v, bias_v,
                *, span, t):
    wid = lax.axis_index("s") * _NC + lax.axis_index("c")
    base = wid * span
    pltpu.sync_copy(bias_hbm, bias_v)
    pltpu.sync_copy(sT.at[:, pl.ds(base, span)], span_v)

    def chunk(c, carry):
        off = c * _L
        v = [span_v[e, pl.ds(off, _L)] for e in range(_E)]
        wvals, ivals = _route_chunk(v, bias_v)
        for r in range(_TOPK):
            wspan_v[r, pl.ds(off, _L)] = wvals[r]
            ispan_v[r, pl.ds(off, _L)] = ivals[r]
        return carry

    lax.fori_loop(0, span // _L, chunk, 0)
    pltpu.sync_copy(wspan_v, wout.at[:, pl.ds(base, span)])
    pltpu.sync_copy(ispan_v, iout.at[:, pl.ds(base, span)])


def _route(s_t, bias):
    t = s_t.shape[1]
    span = t // _NW
    mesh = plsc.VectorSubcoreMesh(
        core_axis_name="c", subcore_axis_name="s",
        num_cores=_NC, num_subcores=_NS)
    fn = pl.kernel(
        functools.partial(_route_body, span=span, t=t),
        out_type=[
            jax.ShapeDtypeStruct((_TOPK, t), jnp.float32),
            jax.ShapeDtypeStruct((_TOPK, t), jnp.int32),
        ],
        mesh=mesh,
        compiler_params=pltpu.CompilerParams(
            use_tc_tiling_on_sc=False, needs_layout_passes=False),
        scratch_types=[
            pltpu.VMEM((_E, span), jnp.float32),
            pltpu.VMEM((_TOPK, span), jnp.float32),
            pltpu.VMEM((_TOPK, span), jnp.int32),
            pltpu.VMEM((_E,), jnp.float32),
        ],
    )
    return fn(s_t, bias)


@jax.jit
def kernel(x, weight, bias):
    e = weight.shape[0]
    w_pad = jnp.pad(weight, ((0, 0), (1, 0)))
    bias_col = bias.reshape(e, 1).astype(jnp.float32)
    s_t = _scores_t(x, w_pad, bias_col)
    w_t, i_t = _route(s_t, bias.astype(jnp.float32))
    return w_t.T, i_t.T
